# baseline (speedup 1.0000x reference)
import functools

import jax
import jax.numpy as jnp
from jax.experimental import pallas as pl
from jax.experimental.pallas import tpu as pltpu


def _conv_body(a_ref, v_ref, vi_ref, w_ref, b_ref, o_ref, *, residual):
    agg = jnp.dot(a_ref[...], v_ref[...], preferred_element_type=jnp.float32)
    vi = vi_ref[...]
    d = vi.shape[1]
    pre = (jnp.dot(vi, w_ref[:d, :], preferred_element_type=jnp.float32)
           + jnp.dot(agg, w_ref[d:, :], preferred_element_type=jnp.float32)
           + b_ref[...])
    h = jnp.maximum(pre, 0.0)
    if residual:
        h = jnp.maximum(h + vi, 0.0)
    o_ref[...] = h


def _graph_conv(v, A, W, b, *, residual, bm):
    n, d = v.shape
    h_dim = W.shape[1]
    return pl.pallas_call(
        functools.partial(_conv_body, residual=residual),
        grid=(n // bm,),
        in_specs=[
            pl.BlockSpec((bm, n), lambda i: (i, 0)),
            pl.BlockSpec((n, d), lambda i: (0, 0)),
            pl.BlockSpec((bm, d), lambda i: (i, 0)),
            pl.BlockSpec((2 * d, h_dim), lambda i: (0, 0)),
            pl.BlockSpec((1, h_dim), lambda i: (0, 0)),
        ],
        out_specs=pl.BlockSpec((bm, h_dim), lambda i: (i, 0)),
        out_shape=jax.ShapeDtypeStruct((n, h_dim), v.dtype),
        compiler_params=pltpu.CompilerParams(
            dimension_semantics=("parallel",),
        ),
    )(A, v, v, W, b.reshape(1, h_dim))


def kernel(x, A, W1, b1, W2, b2):
    bm = 400
    h = _graph_conv(x, A, W1, b1, residual=False, bm=bm)
    out = _graph_conv(h, A, W2, b2, residual=True, bm=bm)
    return out
